# Initial kernel scaffold; baseline (speedup 1.0000x reference)
import jax
import jax.numpy as jnp
from jax.experimental import pallas as pl


def kernel(x, edge_index, W, b, gamma, beta):
    raise NotImplementedError("write your pallas kernel here")


# speedup vs baseline: 7.6159x; 7.6159x over previous
import functools

import jax
import jax.numpy as jnp
from jax import lax
from jax.experimental import pallas as pl
from jax.experimental.pallas import tpu as pltpu
from jax.experimental.pallas import tpu_sc as plsc


def _sc_prep_body(V, E, Ep, ZV, DP,
                  src_hbm, dst_hbm, z_hbm, m_hbm, deg_hbm,
                  mv, degv, srcv, dstv):
    c = lax.axis_index("c")
    s = lax.axis_index("s")

    @pl.when(jnp.logical_and(c == 0, s == 0))
    def _():
        pltpu.sync_copy(z_hbm, mv)
        pltpu.sync_copy(z_hbm.at[pl.ds(0, DP)], degv)
        pltpu.sync_copy(src_hbm, srcv)
        pltpu.sync_copy(dst_hbm, dstv)
        ones = jnp.ones((16,), jnp.float32)
        lane = lax.iota(jnp.int32, 16)

        def edge_body(i, carry):
            base = i * 16
            sv = srcv[pl.ds(base, 16)]
            dv = dstv[pl.ds(base, 16)]
            mask = (base + lane) < E
            idx = dv * V + sv
            plsc.addupdate_scatter(mv, [idx], ones, mask=mask)
            plsc.addupdate_scatter(degv, [dv], ones, mask=mask)
            return carry

        lax.fori_loop(0, Ep // 16, edge_body, 0)

        def diag_body(i, carry):
            base = i * 16
            v = base + lane
            mask = v < V
            idx = v * (V + 1)
            plsc.addupdate_scatter(mv, [idx], ones, mask=mask)
            cur = degv[pl.ds(base, 16)]
            degv[pl.ds(base, 16)] = cur + jnp.where(mask, 1.0, 0.0)
            return carry

        lax.fori_loop(0, DP // 16, diag_body, 0)

        pltpu.sync_copy(mv, m_hbm)
        pltpu.sync_copy(degv, deg_hbm)


def _sc_prep(src_p, dst_p, zeros, V, E, Ep, ZV, DP):
    mesh = plsc.VectorSubcoreMesh(core_axis_name="c", subcore_axis_name="s")
    body = functools.partial(_sc_prep_body, V, E, Ep, ZV, DP)
    return pl.kernel(
        body,
        out_type=(
            jax.ShapeDtypeStruct((ZV,), jnp.float32),
            jax.ShapeDtypeStruct((DP,), jnp.float32),
        ),
        mesh=mesh,
        compiler_params=pltpu.CompilerParams(needs_layout_passes=False),
        scratch_types=[
            pltpu.VMEM((ZV,), jnp.float32),
            pltpu.VMEM((DP,), jnp.float32),
            pltpu.VMEM((Ep,), jnp.int32),
            pltpu.VMEM((Ep,), jnp.int32),
        ],
    )(src_p, dst_p, zeros)


def _gcn_body(nb, N, V, C, x_r, w_r, b_r, m_r, degc_r, degr_r, y_r, st_r, a_s):
    j = pl.program_id(0)

    @pl.when(j == 0)
    def _():
        dinv_c = lax.rsqrt(degc_r[...])
        dinv_r = lax.rsqrt(degr_r[...])
        a_s[...] = m_r[...] * dinv_c * dinv_r
        st_r[...] = jnp.zeros_like(st_r)

    wv = w_r[...]
    bv = b_r[...]
    a = a_s[...]
    s1 = jnp.zeros((V, C), jnp.float32)
    s2 = jnp.zeros((V, C), jnp.float32)
    for g in range(nb):
        xg = x_r[g]
        h = jnp.dot(xg, wv, preferred_element_type=jnp.float32) + bv
        agg = jnp.dot(a, h, preferred_element_type=jnp.float32)
        y = jnp.maximum(agg, 0.0)
        y_r[g] = y
        s1 = s1 + y
        s2 = s2 + y * y
    st_r[0, :, :] = st_r[0, :, :] + s1
    st_r[1, :, :] = st_r[1, :, :] + s2


def _bn_body(nb, N, V, C, y_r, st_r, gamma_r, beta_r, out_r):
    inv_n = 1.0 / N
    mean = st_r[0, :, :] * inv_n
    var = st_r[1, :, :] * inv_n - mean * mean
    rstd = lax.rsqrt(var + 1e-5)
    scale = rstd * gamma_r[...]
    shift = beta_r[...] - mean * scale
    out_r[...] = y_r[...] * scale[None, :, :] + shift[None, :, :]


def kernel(x, edge_index, W, b, gamma, beta):
    B_, T_, V, C = x.shape
    N = B_ * T_
    Co = W.shape[1]
    E = edge_index.shape[1]

    Ep = ((E + 15) // 16) * 16
    ZV = ((V * V + 15) // 16) * 16
    DP = ((V + 15) // 16) * 16

    src = edge_index[0].astype(jnp.int32)
    dst = edge_index[1].astype(jnp.int32)
    pad = jnp.zeros((Ep - E,), jnp.int32)
    src_p = jnp.concatenate([src, pad])
    dst_p = jnp.concatenate([dst, pad])
    zeros = jnp.zeros((ZV,), jnp.float32)

    m_flat, deg_p = _sc_prep(src_p, dst_p, zeros, V, E, Ep, ZV, DP)
    m = m_flat[: V * V].reshape(V, V)
    deg_c = deg_p[:V].reshape(V, 1)
    deg_r = deg_p[:V].reshape(1, V)

    x3 = x.reshape(N, V, C)
    b2 = b.reshape(1, Co)
    gamma2 = gamma.reshape(V, Co)
    beta2 = beta.reshape(V, Co)

    nb = 8
    NB = N // nb

    y, stats = pl.pallas_call(
        functools.partial(_gcn_body, nb, N, V, Co),
        grid=(NB,),
        in_specs=[
            pl.BlockSpec((nb, V, C), lambda j: (j, 0, 0)),
            pl.BlockSpec((C, Co), lambda j: (0, 0)),
            pl.BlockSpec((1, Co), lambda j: (0, 0)),
            pl.BlockSpec((V, V), lambda j: (0, 0)),
            pl.BlockSpec((V, 1), lambda j: (0, 0)),
            pl.BlockSpec((1, V), lambda j: (0, 0)),
        ],
        out_specs=[
            pl.BlockSpec((nb, V, Co), lambda j: (j, 0, 0)),
            pl.BlockSpec((2, V, Co), lambda j: (0, 0, 0)),
        ],
        out_shape=[
            jax.ShapeDtypeStruct((N, V, Co), jnp.float32),
            jax.ShapeDtypeStruct((2, V, Co), jnp.float32),
        ],
        scratch_shapes=[pltpu.VMEM((V, V), jnp.float32)],
    )(x3, W, b2, m, deg_c, deg_r)

    out = pl.pallas_call(
        functools.partial(_bn_body, nb, N, V, Co),
        grid=(NB,),
        in_specs=[
            pl.BlockSpec((nb, V, Co), lambda j: (j, 0, 0)),
            pl.BlockSpec((2, V, Co), lambda j: (0, 0, 0)),
            pl.BlockSpec((V, Co), lambda j: (0, 0)),
            pl.BlockSpec((V, Co), lambda j: (0, 0)),
        ],
        out_specs=pl.BlockSpec((nb, V, Co), lambda j: (j, 0, 0)),
        out_shape=jax.ShapeDtypeStruct((N, V, Co), jnp.float32),
    )(y, stats, gamma2, beta2)

    return out.reshape(B_, T_, V * Co)
